# baseline (speedup 1.0000x reference)
import functools

import jax
import jax.numpy as jnp
from jax import lax
from jax.experimental import pallas as pl
from jax.experimental.pallas import tpu as pltpu
from jax.experimental.pallas import tpu_sc as plsc

N = 10000
E = 160000
D = 256
DH = D // 2

CHUNK = 128
CHUNKS_PER_TILE = 80
E_PAD = 16 * CHUNKS_PER_TILE * CHUNK
ACC_R = 10112
ZROWS = ACC_R // 16
OUT_PT = 624

BN = 1000


def _tc_first_a_body(x_ref, wlT_ref, xl2_ref):
    xl = jnp.dot(x_ref[...], wlT_ref[...], preferred_element_type=jnp.float32)
    xl2_ref[0] = xl[:, :DH]
    xl2_ref[1] = xl[:, DH:]


def _tc_first_b_body(x_ref, wrT_ref, bl_ref, xr_ref):
    xr_ref[...] = jnp.dot(x_ref[...], wrT_ref[...], preferred_element_type=jnp.float32) + bl_ref[...]


def _tc_mid_a_body(agg2_ref, xr_ref, wlT_ref, xl2_ref):
    h = jnp.concatenate([agg2_ref[0], agg2_ref[1]], axis=1) + xr_ref[...]
    h = jnp.maximum(h, 0.0)
    xl = jnp.dot(h, wlT_ref[...], preferred_element_type=jnp.float32)
    xl2_ref[0] = xl[:, :DH]
    xl2_ref[1] = xl[:, DH:]


def _tc_mid_b_body(agg2_ref, xr_ref, wrT_ref, bl_ref, xrn_ref):
    h = jnp.concatenate([agg2_ref[0], agg2_ref[1]], axis=1) + xr_ref[...]
    h = jnp.maximum(h, 0.0)
    xrn_ref[...] = jnp.dot(h, wrT_ref[...], preferred_element_type=jnp.float32) + bl_ref[...]


def _tc_last_body(agg2_ref, xr_ref, out_ref):
    out_ref[...] = jnp.concatenate([agg2_ref[0], agg2_ref[1]], axis=1) + xr_ref[...]


_W_SPEC = pl.BlockSpec((D, D), lambda j: (0, 0))
_B_SPEC = pl.BlockSpec((1, D), lambda j: (0, 0))
_ROW_SPEC = pl.BlockSpec((BN, D), lambda j: (j, 0))
_XL2_SPEC = pl.BlockSpec((2, BN, DH), lambda j: (0, j, 0))

_tc_first_a = pl.pallas_call(
    _tc_first_a_body,
    grid=(N // BN,),
    in_specs=[_ROW_SPEC, _W_SPEC],
    out_specs=_XL2_SPEC,
    out_shape=jax.ShapeDtypeStruct((2, N, DH), jnp.float32),
)

_tc_first_b = pl.pallas_call(
    _tc_first_b_body,
    grid=(N // BN,),
    in_specs=[_ROW_SPEC, _W_SPEC, _B_SPEC],
    out_specs=_ROW_SPEC,
    out_shape=jax.ShapeDtypeStruct((N, D), jnp.float32),
)

_tc_mid_a = pl.pallas_call(
    _tc_mid_a_body,
    grid=(N // BN,),
    in_specs=[_XL2_SPEC, _ROW_SPEC, _W_SPEC],
    out_specs=_XL2_SPEC,
    out_shape=jax.ShapeDtypeStruct((2, N, DH), jnp.float32),
)

_tc_mid_b = pl.pallas_call(
    _tc_mid_b_body,
    grid=(N // BN,),
    in_specs=[_XL2_SPEC, _ROW_SPEC, _W_SPEC, _B_SPEC],
    out_specs=_ROW_SPEC,
    out_shape=jax.ShapeDtypeStruct((N, D), jnp.float32),
)

_tc_last = pl.pallas_call(
    _tc_last_body,
    grid=(N // BN,),
    in_specs=[_XL2_SPEC, _ROW_SPEC],
    out_specs=_ROW_SPEC,
    out_shape=jax.ShapeDtypeStruct((N, D), jnp.float32),
)


def _sc_body(xl_flat, src4, dst3, zeros_hbm, out, sidx_all,
             r0, r1, di0, di1, acc,
             g0, g1, s0, s1, i0, i1):
    c = lax.axis_index("c")
    s = lax.axis_index("s")
    rows = [r0, r1]
    di = [di0, di1]
    gsem = [g0, g1]
    ssem = [s0, s1]
    isem = [i0, i1]

    pltpu.sync_copy(zeros_hbm, r0)
    zbase = s * ZROWS
    for k in range(4):
        pltpu.sync_copy(r0, acc.at[pl.ds(zbase + k * CHUNK, CHUNK)])
    pltpu.sync_copy(r0.at[pl.ds(0, ZROWS - 4 * CHUNK)],
                    acc.at[pl.ds(zbase + 4 * CHUNK, ZROWS - 4 * CHUNK)])

    pltpu.sync_copy(src4.at[c, s], sidx_all)
    for b in range(2):
        pltpu.async_copy(xl_flat.at[sidx_all.at[b]], rows[b], gsem[b])
        pltpu.async_copy(dst3.at[s, b], di[b], isem[b])
    plsc.subcore_barrier()

    def body(j, carry):
        for b in range(2):
            g = 2 * j + b
            pltpu.make_async_copy(xl_flat.at[sidx_all.at[b]], rows[b],
                                  gsem[b]).wait()
            pltpu.make_async_copy(dst3.at[s, b], di[b], isem[b]).wait()
            pltpu.async_copy(rows[b], acc.at[di[b]], ssem[b], add=True)
            pltpu.make_async_copy(rows[b], acc.at[di[b]], ssem[b]).wait()
            pltpu.async_copy(xl_flat.at[sidx_all.at[g + 2]], rows[b],
                             gsem[b])
            pltpu.async_copy(dst3.at[s, g + 2], di[b], isem[b])
        return carry

    lax.fori_loop(0, CHUNKS_PER_TILE // 2 - 1, body, 0)
    for b in range(2):
        pltpu.make_async_copy(xl_flat.at[sidx_all.at[b]], rows[b],
                              gsem[b]).wait()
        pltpu.make_async_copy(dst3.at[s, b], di[b], isem[b]).wait()
        pltpu.async_copy(rows[b], acc.at[di[b]], ssem[b], add=True)
        pltpu.make_async_copy(rows[b], acc.at[di[b]], ssem[b]).wait()
    plsc.subcore_barrier()

    pltpu.sync_copy(acc.at[pl.ds(s * OUT_PT, OUT_PT)],
                    out.at[c, pl.ds(s * OUT_PT, OUT_PT)])

    @pl.when(s == 15)
    def _write_tail():
        pltpu.sync_copy(acc.at[pl.ds(16 * OUT_PT, N - 16 * OUT_PT)],
                        out.at[c, pl.ds(16 * OUT_PT, N - 16 * OUT_PT)])


_sc_segsum = functools.partial(
    pl.kernel,
    out_type=jax.ShapeDtypeStruct((2, N, DH), jnp.float32),
    mesh=plsc.VectorSubcoreMesh(core_axis_name="c", subcore_axis_name="s"),
    scratch_types=[
        pltpu.VMEM((CHUNKS_PER_TILE, CHUNK), jnp.int32),
        pltpu.VMEM((CHUNK, DH), jnp.float32),
        pltpu.VMEM((CHUNK, DH), jnp.float32),
        pltpu.VMEM((CHUNK,), jnp.int32),
        pltpu.VMEM((CHUNK,), jnp.int32),
        pltpu.VMEM_SHARED((ACC_R, DH), jnp.float32),
    ] + [pltpu.SemaphoreType.DMA] * 6,
)(_sc_body)


def kernel(in_feat, edge_index, Wl0, bl0, Wr0, Wl1, bl1, Wr1, Wl2, bl2, Wr2):
    src = edge_index[0].astype(jnp.int32)
    dst = edge_index[1].astype(jnp.int32)
    pad = E_PAD - E
    ar = jnp.arange(pad, dtype=jnp.int32)
    src_p = jnp.concatenate([src, (ar * 37) % N])
    src4 = jnp.stack([src_p, src_p + N]).reshape(2, 16, CHUNKS_PER_TILE, CHUNK)
    dst_p = jnp.concatenate([dst, N + (ar % 16)])
    dst3 = dst_p.reshape(16, CHUNKS_PER_TILE, CHUNK)
    zeros = jnp.zeros((CHUNK, DH), jnp.float32)

    def layer_agg(xl2):
        return _sc_segsum(xl2.reshape(2 * N, DH), src4, dst3, zeros)

    xl2 = _tc_first_a(in_feat, Wl0.T)
    agg2 = layer_agg(xl2)
    xr = _tc_first_b(in_feat, Wr0.T, bl0.reshape(1, D))
    xl2 = _tc_mid_a(agg2, xr, Wl1.T)
    nagg2 = layer_agg(xl2)
    xr = _tc_mid_b(agg2, xr, Wr1.T, bl1.reshape(1, D))
    agg2 = nagg2
    xl2 = _tc_mid_a(agg2, xr, Wl2.T)
    nagg2 = layer_agg(xl2)
    xr = _tc_mid_b(agg2, xr, Wr2.T, bl2.reshape(1, D))
    return _tc_last(nagg2, xr)


# speedup vs baseline: 1.0074x; 1.0074x over previous
import functools

import jax
import jax.numpy as jnp
from jax import lax
from jax.experimental import pallas as pl
from jax.experimental.pallas import tpu as pltpu
from jax.experimental.pallas import tpu_sc as plsc

N = 10000
E = 160000
D = 256
DH = D // 2

CHUNK = 128
CHUNKS_PER_TILE = 80
E_PAD = 16 * CHUNKS_PER_TILE * CHUNK
ACC_R = 10112
ZROWS = ACC_R // 16
OUT_PT = 624

BN = 1000


def _tc_first_body(x_ref, wlT_ref, wrT_ref, bl_ref, xl2_ref, xr_ref):
    h = x_ref[...]
    xl = jnp.dot(h, wlT_ref[...], preferred_element_type=jnp.float32)
    xl2_ref[0] = xl[:, :DH]
    xl2_ref[1] = xl[:, DH:]
    xr_ref[...] = jnp.dot(h, wrT_ref[...], preferred_element_type=jnp.float32) + bl_ref[...]


def _tc_mid_body(agg2_ref, xr_ref, wlT_ref, wrT_ref, bl_ref, xl2_ref, xrn_ref):
    h = jnp.concatenate([agg2_ref[0], agg2_ref[1]], axis=1) + xr_ref[...]
    h = jnp.maximum(h, 0.0)
    xl = jnp.dot(h, wlT_ref[...], preferred_element_type=jnp.float32)
    xl2_ref[0] = xl[:, :DH]
    xl2_ref[1] = xl[:, DH:]
    xrn_ref[...] = jnp.dot(h, wrT_ref[...], preferred_element_type=jnp.float32) + bl_ref[...]


def _tc_last_body(agg2_ref, xr_ref, out_ref):
    out_ref[...] = jnp.concatenate([agg2_ref[0], agg2_ref[1]], axis=1) + xr_ref[...]


_W_SPEC = pl.BlockSpec((D, D), lambda j: (0, 0))
_B_SPEC = pl.BlockSpec((1, D), lambda j: (0, 0))
_ROW_SPEC = pl.BlockSpec((BN, D), lambda j: (j, 0))
_XL2_SPEC = pl.BlockSpec((2, BN, DH), lambda j: (0, j, 0))

_tc_first = pl.pallas_call(
    _tc_first_body,
    grid=(N // BN,),
    in_specs=[_ROW_SPEC, _W_SPEC, _W_SPEC, _B_SPEC],
    out_specs=[_XL2_SPEC, _ROW_SPEC],
    out_shape=[
        jax.ShapeDtypeStruct((2, N, DH), jnp.float32),
        jax.ShapeDtypeStruct((N, D), jnp.float32),
    ],
)

_tc_mid = pl.pallas_call(
    _tc_mid_body,
    grid=(N // BN,),
    in_specs=[_XL2_SPEC, _ROW_SPEC, _W_SPEC, _W_SPEC, _B_SPEC],
    out_specs=[_XL2_SPEC, _ROW_SPEC],
    out_shape=[
        jax.ShapeDtypeStruct((2, N, DH), jnp.float32),
        jax.ShapeDtypeStruct((N, D), jnp.float32),
    ],
)

_tc_last = pl.pallas_call(
    _tc_last_body,
    grid=(N // BN,),
    in_specs=[_XL2_SPEC, _ROW_SPEC],
    out_specs=_ROW_SPEC,
    out_shape=jax.ShapeDtypeStruct((N, D), jnp.float32),
)


def _sc_body(xl_flat, src4, dst3, zeros_hbm, out, sidx_all,
             r0, r1, di0, di1, acc,
             g0, g1, s0, s1, i0, i1):
    c = lax.axis_index("c")
    s = lax.axis_index("s")
    rows = [r0, r1]
    di = [di0, di1]
    gsem = [g0, g1]
    ssem = [s0, s1]
    isem = [i0, i1]

    pltpu.sync_copy(zeros_hbm, r0)
    zbase = s * ZROWS
    for k in range(4):
        pltpu.sync_copy(r0, acc.at[pl.ds(zbase + k * CHUNK, CHUNK)])
    pltpu.sync_copy(r0.at[pl.ds(0, ZROWS - 4 * CHUNK)],
                    acc.at[pl.ds(zbase + 4 * CHUNK, ZROWS - 4 * CHUNK)])

    pltpu.sync_copy(src4.at[c, s], sidx_all)
    for b in range(2):
        pltpu.async_copy(xl_flat.at[sidx_all.at[b]], rows[b], gsem[b])
        pltpu.async_copy(dst3.at[s, b], di[b], isem[b])
    plsc.subcore_barrier()

    def body(j, carry):
        for b in range(2):
            g = 2 * j + b
            pltpu.make_async_copy(xl_flat.at[sidx_all.at[b]], rows[b],
                                  gsem[b]).wait()
            pltpu.make_async_copy(dst3.at[s, b], di[b], isem[b]).wait()
            pltpu.async_copy(rows[b], acc.at[di[b]], ssem[b], add=True)
            pltpu.make_async_copy(rows[b], acc.at[di[b]], ssem[b]).wait()
            pltpu.async_copy(xl_flat.at[sidx_all.at[g + 2]], rows[b],
                             gsem[b])
            pltpu.async_copy(dst3.at[s, g + 2], di[b], isem[b])
        return carry

    lax.fori_loop(0, CHUNKS_PER_TILE // 2 - 1, body, 0)
    for b in range(2):
        pltpu.make_async_copy(xl_flat.at[sidx_all.at[b]], rows[b],
                              gsem[b]).wait()
        pltpu.make_async_copy(dst3.at[s, b], di[b], isem[b]).wait()
        pltpu.async_copy(rows[b], acc.at[di[b]], ssem[b], add=True)
        pltpu.make_async_copy(rows[b], acc.at[di[b]], ssem[b]).wait()
    plsc.subcore_barrier()

    pltpu.sync_copy(acc.at[pl.ds(s * OUT_PT, OUT_PT)],
                    out.at[c, pl.ds(s * OUT_PT, OUT_PT)])

    @pl.when(s == 15)
    def _write_tail():
        pltpu.sync_copy(acc.at[pl.ds(16 * OUT_PT, N - 16 * OUT_PT)],
                        out.at[c, pl.ds(16 * OUT_PT, N - 16 * OUT_PT)])


_sc_segsum = functools.partial(
    pl.kernel,
    out_type=jax.ShapeDtypeStruct((2, N, DH), jnp.float32),
    mesh=plsc.VectorSubcoreMesh(core_axis_name="c", subcore_axis_name="s"),
    scratch_types=[
        pltpu.VMEM((CHUNKS_PER_TILE, CHUNK), jnp.int32),
        pltpu.VMEM((CHUNK, DH), jnp.float32),
        pltpu.VMEM((CHUNK, DH), jnp.float32),
        pltpu.VMEM((CHUNK,), jnp.int32),
        pltpu.VMEM((CHUNK,), jnp.int32),
        pltpu.VMEM_SHARED((ACC_R, DH), jnp.float32),
    ] + [pltpu.SemaphoreType.DMA] * 6,
)(_sc_body)


def kernel(in_feat, edge_index, Wl0, bl0, Wr0, Wl1, bl1, Wr1, Wl2, bl2, Wr2):
    src = edge_index[0].astype(jnp.int32)
    dst = edge_index[1].astype(jnp.int32)
    pad = E_PAD - E
    ar = jnp.arange(pad, dtype=jnp.int32)
    src_p = jnp.concatenate([src, (ar * 37) % N])
    src4 = jnp.stack([src_p, src_p + N]).reshape(2, 16, CHUNKS_PER_TILE, CHUNK)
    dst_p = jnp.concatenate([dst, N + (ar % 16)])
    dst3 = dst_p.reshape(16, CHUNKS_PER_TILE, CHUNK)
    zeros = jnp.zeros((CHUNK, DH), jnp.float32)

    def layer_agg(xl2):
        return _sc_segsum(xl2.reshape(2 * N, DH), src4, dst3, zeros)

    xl2, xr = _tc_first(in_feat, Wl0.T, Wr0.T, bl0.reshape(1, D))
    agg2 = layer_agg(xl2)
    xl2, xr = _tc_mid(agg2, xr, Wl1.T, Wr1.T, bl1.reshape(1, D))
    agg2 = layer_agg(xl2)
    xl2, xr = _tc_mid(agg2, xr, Wl2.T, Wr2.T, bl2.reshape(1, D))
    agg2 = layer_agg(xl2)
    return _tc_last(agg2, xr)


# speedup vs baseline: 1.0251x; 1.0176x over previous
import functools

import jax
import jax.numpy as jnp
from jax import lax
from jax.experimental import pallas as pl
from jax.experimental.pallas import tpu as pltpu
from jax.experimental.pallas import tpu_sc as plsc

N = 10000
E = 160000
D = 256
DH = D // 2

CHUNK = 128
CHUNKS_PER_TILE = 80
E_PAD = 16 * CHUNKS_PER_TILE * CHUNK
ACC_R = 10112
ZROWS = ACC_R // 16
OUT_PT = 624

BN = 2000


def _tc_first_body(x_ref, wlT_ref, wrT_ref, bl_ref, xl2_ref, xr_ref):
    h = x_ref[...]
    xl = jnp.dot(h, wlT_ref[...], preferred_element_type=jnp.float32)
    xl2_ref[0] = xl[:, :DH]
    xl2_ref[1] = xl[:, DH:]
    xr_ref[...] = jnp.dot(h, wrT_ref[...], preferred_element_type=jnp.float32) + bl_ref[...]


def _tc_mid_body(agg2_ref, xr_ref, wlT_ref, wrT_ref, bl_ref, xl2_ref, xrn_ref):
    h = jnp.concatenate([agg2_ref[0], agg2_ref[1]], axis=1) + xr_ref[...]
    h = jnp.maximum(h, 0.0)
    xl = jnp.dot(h, wlT_ref[...], preferred_element_type=jnp.float32)
    xl2_ref[0] = xl[:, :DH]
    xl2_ref[1] = xl[:, DH:]
    xrn_ref[...] = jnp.dot(h, wrT_ref[...], preferred_element_type=jnp.float32) + bl_ref[...]


def _tc_last_body(agg2_ref, xr_ref, out_ref):
    out_ref[...] = jnp.concatenate([agg2_ref[0], agg2_ref[1]], axis=1) + xr_ref[...]


_W_SPEC = pl.BlockSpec((D, D), lambda j: (0, 0))
_B_SPEC = pl.BlockSpec((1, D), lambda j: (0, 0))
_ROW_SPEC = pl.BlockSpec((BN, D), lambda j: (j, 0))
_XL2_SPEC = pl.BlockSpec((2, BN, DH), lambda j: (0, j, 0))

_tc_first = pl.pallas_call(
    _tc_first_body,
    grid=(N // BN,),
    in_specs=[_ROW_SPEC, _W_SPEC, _W_SPEC, _B_SPEC],
    out_specs=[_XL2_SPEC, _ROW_SPEC],
    out_shape=[
        jax.ShapeDtypeStruct((2, N, DH), jnp.float32),
        jax.ShapeDtypeStruct((N, D), jnp.float32),
    ],
)

_tc_mid = pl.pallas_call(
    _tc_mid_body,
    grid=(N // BN,),
    in_specs=[_XL2_SPEC, _ROW_SPEC, _W_SPEC, _W_SPEC, _B_SPEC],
    out_specs=[_XL2_SPEC, _ROW_SPEC],
    out_shape=[
        jax.ShapeDtypeStruct((2, N, DH), jnp.float32),
        jax.ShapeDtypeStruct((N, D), jnp.float32),
    ],
)

_tc_last = pl.pallas_call(
    _tc_last_body,
    grid=(N // BN,),
    in_specs=[_XL2_SPEC, _ROW_SPEC],
    out_specs=_ROW_SPEC,
    out_shape=jax.ShapeDtypeStruct((N, D), jnp.float32),
)


def _sc_body(xl_flat, src4, dst3, zeros_hbm, out, sidx_all,
             r0, r1, di0, di1, acc,
             g0, g1, s0, s1, i0, i1):
    c = lax.axis_index("c")
    s = lax.axis_index("s")
    rows = [r0, r1]
    di = [di0, di1]
    gsem = [g0, g1]
    ssem = [s0, s1]
    isem = [i0, i1]

    pltpu.sync_copy(zeros_hbm, r0)
    zbase = s * ZROWS
    for k in range(4):
        pltpu.sync_copy(r0, acc.at[pl.ds(zbase + k * CHUNK, CHUNK)])
    pltpu.sync_copy(r0.at[pl.ds(0, ZROWS - 4 * CHUNK)],
                    acc.at[pl.ds(zbase + 4 * CHUNK, ZROWS - 4 * CHUNK)])

    pltpu.sync_copy(src4.at[c, s], sidx_all)
    for b in range(2):
        pltpu.async_copy(xl_flat.at[sidx_all.at[b]], rows[b], gsem[b])
        pltpu.async_copy(dst3.at[s, b], di[b], isem[b])
    plsc.subcore_barrier()

    def body(j, carry):
        for b in range(2):
            g = 2 * j + b
            pltpu.make_async_copy(xl_flat.at[sidx_all.at[b]], rows[b],
                                  gsem[b]).wait()
            pltpu.make_async_copy(dst3.at[s, b], di[b], isem[b]).wait()
            pltpu.async_copy(rows[b], acc.at[di[b]], ssem[b], add=True)
            pltpu.make_async_copy(rows[b], acc.at[di[b]], ssem[b]).wait()
            pltpu.async_copy(xl_flat.at[sidx_all.at[g + 2]], rows[b],
                             gsem[b])
            pltpu.async_copy(dst3.at[s, g + 2], di[b], isem[b])
        return carry

    lax.fori_loop(0, CHUNKS_PER_TILE // 2 - 1, body, 0)
    for b in range(2):
        pltpu.make_async_copy(xl_flat.at[sidx_all.at[b]], rows[b],
                              gsem[b]).wait()
        pltpu.make_async_copy(dst3.at[s, b], di[b], isem[b]).wait()
        pltpu.async_copy(rows[b], acc.at[di[b]], ssem[b], add=True)
        pltpu.make_async_copy(rows[b], acc.at[di[b]], ssem[b]).wait()
    plsc.subcore_barrier()

    pltpu.sync_copy(acc.at[pl.ds(s * OUT_PT, OUT_PT)],
                    out.at[c, pl.ds(s * OUT_PT, OUT_PT)])

    @pl.when(s == 15)
    def _write_tail():
        pltpu.sync_copy(acc.at[pl.ds(16 * OUT_PT, N - 16 * OUT_PT)],
                        out.at[c, pl.ds(16 * OUT_PT, N - 16 * OUT_PT)])


_sc_segsum = functools.partial(
    pl.kernel,
    out_type=jax.ShapeDtypeStruct((2, N, DH), jnp.float32),
    mesh=plsc.VectorSubcoreMesh(core_axis_name="c", subcore_axis_name="s"),
    scratch_types=[
        pltpu.VMEM((CHUNKS_PER_TILE, CHUNK), jnp.int32),
        pltpu.VMEM((CHUNK, DH), jnp.float32),
        pltpu.VMEM((CHUNK, DH), jnp.float32),
        pltpu.VMEM((CHUNK,), jnp.int32),
        pltpu.VMEM((CHUNK,), jnp.int32),
        pltpu.VMEM_SHARED((ACC_R, DH), jnp.float32),
    ] + [pltpu.SemaphoreType.DMA] * 6,
)(_sc_body)


def kernel(in_feat, edge_index, Wl0, bl0, Wr0, Wl1, bl1, Wr1, Wl2, bl2, Wr2):
    src = edge_index[0].astype(jnp.int32)
    dst = edge_index[1].astype(jnp.int32)
    pad = E_PAD - E
    ar = jnp.arange(pad, dtype=jnp.int32)
    src_p = jnp.concatenate([src, (ar * 37) % N])
    src4 = jnp.stack([src_p, src_p + N]).reshape(2, 16, CHUNKS_PER_TILE, CHUNK)
    dst_p = jnp.concatenate([dst, N + (ar % 16)])
    dst3 = dst_p.reshape(16, CHUNKS_PER_TILE, CHUNK)
    zeros = jnp.zeros((CHUNK, DH), jnp.float32)

    def layer_agg(xl2):
        return _sc_segsum(xl2.reshape(2 * N, DH), src4, dst3, zeros)

    xl2, xr = _tc_first(in_feat, Wl0.T, Wr0.T, bl0.reshape(1, D))
    agg2 = layer_agg(xl2)
    xl2, xr = _tc_mid(agg2, xr, Wl1.T, Wr1.T, bl1.reshape(1, D))
    agg2 = layer_agg(xl2)
    xl2, xr = _tc_mid(agg2, xr, Wl2.T, Wr2.T, bl2.reshape(1, D))
    agg2 = layer_agg(xl2)
    return _tc_last(agg2, xr)


# speedup vs baseline: 1.0419x; 1.0163x over previous
import functools

import jax
import jax.numpy as jnp
from jax import lax
from jax.experimental import pallas as pl
from jax.experimental.pallas import tpu as pltpu
from jax.experimental.pallas import tpu_sc as plsc

N = 10000
E = 160000
D = 256
DH = D // 2

CHUNK = 128
CHUNKS_PER_TILE = 80
E_PAD = 16 * CHUNKS_PER_TILE * CHUNK
ACC_R = 10112
ZROWS = ACC_R // 16
OUT_PT = 624

BN = 5000


def _tc_first_body(x_ref, wlT_ref, wrT_ref, bl_ref, xl2_ref, xr_ref):
    h = x_ref[...]
    xl = jnp.dot(h, wlT_ref[...], preferred_element_type=jnp.float32)
    xl2_ref[0] = xl[:, :DH]
    xl2_ref[1] = xl[:, DH:]
    xr_ref[...] = jnp.dot(h, wrT_ref[...], preferred_element_type=jnp.float32) + bl_ref[...]


def _tc_mid_body(agg2_ref, xr_ref, wlT_ref, wrT_ref, bl_ref, xl2_ref, xrn_ref):
    h = jnp.concatenate([agg2_ref[0], agg2_ref[1]], axis=1) + xr_ref[...]
    h = jnp.maximum(h, 0.0)
    xl = jnp.dot(h, wlT_ref[...], preferred_element_type=jnp.float32)
    xl2_ref[0] = xl[:, :DH]
    xl2_ref[1] = xl[:, DH:]
    xrn_ref[...] = jnp.dot(h, wrT_ref[...], preferred_element_type=jnp.float32) + bl_ref[...]


def _tc_last_body(agg2_ref, xr_ref, out_ref):
    out_ref[...] = jnp.concatenate([agg2_ref[0], agg2_ref[1]], axis=1) + xr_ref[...]


_W_SPEC = pl.BlockSpec((D, D), lambda j: (0, 0))
_B_SPEC = pl.BlockSpec((1, D), lambda j: (0, 0))
_ROW_SPEC = pl.BlockSpec((BN, D), lambda j: (j, 0))
_XL2_SPEC = pl.BlockSpec((2, BN, DH), lambda j: (0, j, 0))

_tc_first = pl.pallas_call(
    _tc_first_body,
    grid=(N // BN,),
    in_specs=[_ROW_SPEC, _W_SPEC, _W_SPEC, _B_SPEC],
    out_specs=[_XL2_SPEC, _ROW_SPEC],
    out_shape=[
        jax.ShapeDtypeStruct((2, N, DH), jnp.float32),
        jax.ShapeDtypeStruct((N, D), jnp.float32),
    ],
)

_tc_mid = pl.pallas_call(
    _tc_mid_body,
    grid=(N // BN,),
    in_specs=[_XL2_SPEC, _ROW_SPEC, _W_SPEC, _W_SPEC, _B_SPEC],
    out_specs=[_XL2_SPEC, _ROW_SPEC],
    out_shape=[
        jax.ShapeDtypeStruct((2, N, DH), jnp.float32),
        jax.ShapeDtypeStruct((N, D), jnp.float32),
    ],
)

_tc_last = pl.pallas_call(
    _tc_last_body,
    grid=(N // BN,),
    in_specs=[_XL2_SPEC, _ROW_SPEC],
    out_specs=_ROW_SPEC,
    out_shape=jax.ShapeDtypeStruct((N, D), jnp.float32),
)


def _sc_body(xl_flat, src4, dst3, zeros_hbm, out, sidx_all,
             r0, r1, di0, di1, acc,
             g0, g1, s0, s1, i0, i1):
    c = lax.axis_index("c")
    s = lax.axis_index("s")
    rows = [r0, r1]
    di = [di0, di1]
    gsem = [g0, g1]
    ssem = [s0, s1]
    isem = [i0, i1]

    pltpu.sync_copy(zeros_hbm, r0)
    zbase = s * ZROWS
    for k in range(4):
        pltpu.sync_copy(r0, acc.at[pl.ds(zbase + k * CHUNK, CHUNK)])
    pltpu.sync_copy(r0.at[pl.ds(0, ZROWS - 4 * CHUNK)],
                    acc.at[pl.ds(zbase + 4 * CHUNK, ZROWS - 4 * CHUNK)])

    pltpu.sync_copy(src4.at[c, s], sidx_all)
    for b in range(2):
        pltpu.async_copy(xl_flat.at[sidx_all.at[b]], rows[b], gsem[b])
        pltpu.async_copy(dst3.at[s, b], di[b], isem[b])
    plsc.subcore_barrier()

    def body(j, carry):
        for b in range(2):
            g = 2 * j + b
            pltpu.make_async_copy(xl_flat.at[sidx_all.at[b]], rows[b],
                                  gsem[b]).wait()
            pltpu.make_async_copy(dst3.at[s, b], di[b], isem[b]).wait()
            pltpu.async_copy(rows[b], acc.at[di[b]], ssem[b], add=True)
            pltpu.make_async_copy(rows[b], acc.at[di[b]], ssem[b]).wait()
            pltpu.async_copy(xl_flat.at[sidx_all.at[g + 2]], rows[b],
                             gsem[b])
            pltpu.async_copy(dst3.at[s, g + 2], di[b], isem[b])
        return carry

    lax.fori_loop(0, CHUNKS_PER_TILE // 2 - 1, body, 0)
    for b in range(2):
        pltpu.make_async_copy(xl_flat.at[sidx_all.at[b]], rows[b],
                              gsem[b]).wait()
        pltpu.make_async_copy(dst3.at[s, b], di[b], isem[b]).wait()
        pltpu.async_copy(rows[b], acc.at[di[b]], ssem[b], add=True)
        pltpu.make_async_copy(rows[b], acc.at[di[b]], ssem[b]).wait()
    plsc.subcore_barrier()

    pltpu.sync_copy(acc.at[pl.ds(s * OUT_PT, OUT_PT)],
                    out.at[c, pl.ds(s * OUT_PT, OUT_PT)])

    @pl.when(s == 15)
    def _write_tail():
        pltpu.sync_copy(acc.at[pl.ds(16 * OUT_PT, N - 16 * OUT_PT)],
                        out.at[c, pl.ds(16 * OUT_PT, N - 16 * OUT_PT)])


_sc_segsum = functools.partial(
    pl.kernel,
    out_type=jax.ShapeDtypeStruct((2, N, DH), jnp.float32),
    mesh=plsc.VectorSubcoreMesh(core_axis_name="c", subcore_axis_name="s"),
    scratch_types=[
        pltpu.VMEM((CHUNKS_PER_TILE, CHUNK), jnp.int32),
        pltpu.VMEM((CHUNK, DH), jnp.float32),
        pltpu.VMEM((CHUNK, DH), jnp.float32),
        pltpu.VMEM((CHUNK,), jnp.int32),
        pltpu.VMEM((CHUNK,), jnp.int32),
        pltpu.VMEM_SHARED((ACC_R, DH), jnp.float32),
    ] + [pltpu.SemaphoreType.DMA] * 6,
)(_sc_body)


def kernel(in_feat, edge_index, Wl0, bl0, Wr0, Wl1, bl1, Wr1, Wl2, bl2, Wr2):
    src = edge_index[0].astype(jnp.int32)
    dst = edge_index[1].astype(jnp.int32)
    pad = E_PAD - E
    ar = jnp.arange(pad, dtype=jnp.int32)
    src_p = jnp.concatenate([src, (ar * 37) % N])
    src4 = jnp.stack([src_p, src_p + N]).reshape(2, 16, CHUNKS_PER_TILE, CHUNK)
    dst_p = jnp.concatenate([dst, N + (ar % 16)])
    dst3 = dst_p.reshape(16, CHUNKS_PER_TILE, CHUNK)
    zeros = jnp.zeros((CHUNK, DH), jnp.float32)

    def layer_agg(xl2):
        return _sc_segsum(xl2.reshape(2 * N, DH), src4, dst3, zeros)

    xl2, xr = _tc_first(in_feat, Wl0.T, Wr0.T, bl0.reshape(1, D))
    agg2 = layer_agg(xl2)
    xl2, xr = _tc_mid(agg2, xr, Wl1.T, Wr1.T, bl1.reshape(1, D))
    agg2 = layer_agg(xl2)
    xl2, xr = _tc_mid(agg2, xr, Wl2.T, Wr2.T, bl2.reshape(1, D))
    agg2 = layer_agg(xl2)
    return _tc_last(agg2, xr)
